# baseline (speedup 1.0000x reference)
import functools

import jax
import jax.numpy as jnp
from jax import lax
from jax.experimental import pallas as pl
from jax.experimental.pallas import tpu as pltpu
from jax.experimental.pallas import tpu_sc as plsc

N_NODES = 10000
D_FEAT = 128
N_EDGES = 320000

NC = 2
NS = 16
NW = NC * NS
CK = 128
NCH = 80
E_PAD = NW * NCH * CK
N_PAD = 10240
ROWS_PER_TILE = N_PAD // NS


NBUF = 2


def _layer_body(table, srcs, dsts, ews, zeros, part,
                acc, dst_v, sring, ewring, rows, gsem, esem):
    c = lax.axis_index("c")
    s = lax.axis_index("s")
    wid = c * NS + s

    pltpu.sync_copy(dsts.at[wid], dst_v)

    pltpu.sync_copy(zeros, acc.at[pl.ds(s * ROWS_PER_TILE, ROWS_PER_TILE)])
    plsc.subcore_barrier()

    def _echunk(jj):
        return pl.ds((wid * NCH + jj) * CK, CK)

    def edge_fetch(jj, slot):
        return (
            pltpu.async_copy(srcs.at[_echunk(jj)], sring[slot],
                             esem[slot]),
            pltpu.async_copy(ews.at[_echunk(jj)], ewring[slot],
                             esem[slot]),
        )

    def scale(b, rb):
        def rowgrp(rbk, carry2):
            for i in range(16):
                r = rbk * 16 + i
                w = plsc.load_gather(
                    ewring[rb], [jnp.full((16,), r, jnp.int32)])
                for cb in range(8):
                    sl = rows[rb][r, pl.ds(cb * 16, 16)]
                    rows[rb][r, pl.ds(cb * 16, 16)] = sl * w
            return carry2

        lax.fori_loop(0, 8, rowgrp, 0)

    def stage(j, b, do_next):
        rb = b % NBUF
        nb = (b + 1) % NBUF

        if do_next:
            edescs = edge_fetch(j + 1, nb)
        scale(b, rb)
        if do_next:
            for d in edescs:
                d.wait()
            gdesc = pltpu.async_copy(table.at[sring[nb]], rows[nb],
                                     gsem[nb])
        pltpu.sync_copy(rows[rb], acc.at[dst_v.at[jnp.asarray(j, jnp.int32)]],
                        add=True)
        if do_next:
            gdesc.wait()

    for d in edge_fetch(0, 0):
        d.wait()
    pltpu.async_copy(table.at[sring[0]], rows[0], gsem[0]).wait()

    def group(g, carry):
        for b in range(2):
            stage(2 * g + b, b, do_next=True)
        return carry

    lax.fori_loop(0, NCH // 2 - 1, group, 0)

    stage(NCH - 2, 0, do_next=True)
    stage(NCH - 1, 1, do_next=False)

    plsc.subcore_barrier()

    pltpu.sync_copy(acc.at[pl.ds(s * ROWS_PER_TILE, ROWS_PER_TILE)],
                    part.at[c, pl.ds(s * ROWS_PER_TILE, ROWS_PER_TILE)])


@functools.cache
def _get_layer():
    return pl.kernel(
        _layer_body,
        out_type=jax.ShapeDtypeStruct((NC, N_PAD, D_FEAT), jnp.float32),
        mesh=plsc.VectorSubcoreMesh(core_axis_name="c", subcore_axis_name="s",
                                    num_cores=NC, num_subcores=NS),
        compiler_params=pltpu.CompilerParams(needs_layout_passes=False),
        scratch_types=[
            pltpu.VMEM_SHARED((N_PAD, D_FEAT), jnp.float32),
            pltpu.VMEM((NCH, CK), jnp.int32),
            [pltpu.VMEM((CK,), jnp.int32)] * NBUF,
            [pltpu.VMEM((CK,), jnp.float32)] * NBUF,
            [pltpu.VMEM((CK, D_FEAT), jnp.float32)] * NBUF,
            [pltpu.SemaphoreType.DMA] * NBUF,
            [pltpu.SemaphoreType.DMA] * NBUF,
        ],
    )


def _combine_body(p_ref, o_ref):
    o_ref[...] = p_ref[0] + p_ref[1]


def _final_body(x_ref, h1_ref, p_ref, o_ref):
    o_ref[...] = (x_ref[...] + h1_ref[...] + p_ref[0] + p_ref[1]) * (1.0 / 3.0)


_GB = 1000

_combine = pl.pallas_call(
    _combine_body,
    grid=(N_NODES // _GB,),
    in_specs=[pl.BlockSpec((NC, _GB, D_FEAT), lambda i: (0, i, 0))],
    out_specs=pl.BlockSpec((_GB, D_FEAT), lambda i: (i, 0)),
    out_shape=jax.ShapeDtypeStruct((N_NODES, D_FEAT), jnp.float32),
)

_final = pl.pallas_call(
    _final_body,
    grid=(N_NODES // _GB,),
    in_specs=[
        pl.BlockSpec((_GB, D_FEAT), lambda i: (i, 0)),
        pl.BlockSpec((_GB, D_FEAT), lambda i: (i, 0)),
        pl.BlockSpec((NC, _GB, D_FEAT), lambda i: (0, i, 0)),
    ],
    out_specs=pl.BlockSpec((_GB, D_FEAT), lambda i: (i, 0)),
    out_shape=jax.ShapeDtypeStruct((N_NODES, D_FEAT), jnp.float32),
)


def kernel(x, edge_index, edge_weight):
    src = edge_index[0].astype(jnp.int32)
    dst = edge_index[1].astype(jnp.int32)
    ew = edge_weight.astype(jnp.float32)

    pad = E_PAD - N_EDGES
    src = jnp.pad(src, (0, pad))
    dst = jnp.pad(dst, (0, pad)).reshape(NW, NCH, CK)
    ew = jnp.pad(ew, (0, pad))

    zeros = jnp.zeros((ROWS_PER_TILE, D_FEAT), jnp.float32)

    layer = _get_layer()
    p1 = layer(x, src, dst, ew, zeros)
    h1 = _combine(p1)
    p2 = layer(h1, src, dst, ew, zeros)
    return _final(x, h1, p2)


# speedup vs baseline: 1.1926x; 1.1926x over previous
import functools

import jax
import jax.numpy as jnp
from jax import lax
from jax.experimental import pallas as pl
from jax.experimental.pallas import tpu as pltpu
from jax.experimental.pallas import tpu_sc as plsc

N_NODES = 10000
D_FEAT = 128
N_EDGES = 320000

NC = 2
NS = 16
NW = NC * NS
CK = 128
NCH = 80
E_PAD = NW * NCH * CK
N_PAD = 10240
ROWS_PER_TILE = N_PAD // NS


NBUF = 2
EDEPTH = 4


def _layer_body(table, srcs, dsts, ews, zeros, part,
                acc, dst_v, sring, ewring, rows, gsem, ssem, esem):
    c = lax.axis_index("c")
    s = lax.axis_index("s")
    wid = c * NS + s

    pltpu.sync_copy(dsts.at[wid], dst_v)

    pltpu.sync_copy(zeros, acc.at[pl.ds(s * ROWS_PER_TILE, ROWS_PER_TILE)])
    plsc.subcore_barrier()

    def _echunk(jj):
        return pl.ds((wid * NCH + jj) * CK, CK)

    def edge_fetch(jj, slot):
        pltpu.async_copy(srcs.at[_echunk(jj)], sring[slot], esem[slot])
        pltpu.async_copy(ews.at[_echunk(jj)], ewring[slot], esem[slot])

    def edge_wait(jj, slot):
        pltpu.make_async_copy(srcs.at[_echunk(jj)], sring[slot],
                              esem[slot]).wait()
        pltpu.make_async_copy(ews.at[_echunk(jj)], ewring[slot],
                              esem[slot]).wait()

    def gather_start(slot, rb):
        pltpu.async_copy(table.at[sring[slot]], rows[rb], gsem[rb])

    def gather_wait(slot, rb):
        pltpu.make_async_copy(table.at[sring[slot]], rows[rb],
                              gsem[rb]).wait()

    def scatter_start(j, rb):
        pltpu.async_copy(rows[rb], acc.at[dst_v.at[jnp.asarray(j, jnp.int32)]],
                        ssem[rb], add=True)

    def scatter_wait(rb):
        pltpu.make_async_copy(rows[rb], acc.at[dst_v.at[0]],
                              ssem[rb]).wait()

    def scale(b, rb):
        def rowgrp(rbk, carry2):
            for i in range(16):
                r = rbk * 16 + i
                w = plsc.load_gather(
                    ewring[b], [jnp.full((16,), r, jnp.int32)])
                for cb in range(8):
                    sl = rows[rb][r, pl.ds(cb * 16, 16)]
                    rows[rb][r, pl.ds(cb * 16, 16)] = sl * w
            return carry2

        lax.fori_loop(0, 8, rowgrp, 0)

    def stage(j, b, drain, gather_next, fetch):
        rb = b % NBUF
        nrb = (b + 1) % NBUF
        nslot = (b + 1) % EDEPTH
        fslot = (b + 3) % EDEPTH

        if drain:
            scatter_wait(nrb)
        if gather_next:
            edge_wait(j + 1, nslot)
            gather_start(nslot, nrb)
        if fetch:
            edge_fetch(j + 3, fslot)

        gather_wait(b, rb)
        scale(b, rb)
        scatter_start(j, rb)

    for jc in range(EDEPTH - 1):
        edge_fetch(jc, jc)
    edge_wait(0, 0)
    gather_start(0, 0)

    for j in range(EDEPTH):
        stage(j, j, drain=(j >= 1), gather_next=True, fetch=True)

    def group(g, carry):
        for b in range(EDEPTH):
            stage(EDEPTH * g + b, b, drain=True, gather_next=True,
                  fetch=True)
        return carry

    lax.fori_loop(1, NCH // EDEPTH - 1, group, 0)

    for j in range(NCH - EDEPTH, NCH):
        stage(j, j % EDEPTH, drain=(j + 1 < NCH),
              gather_next=(j + 1 < NCH), fetch=(j + 3 < NCH))

    scatter_wait(0)
    scatter_wait(1)

    plsc.subcore_barrier()

    pltpu.sync_copy(acc.at[pl.ds(s * ROWS_PER_TILE, ROWS_PER_TILE)],
                    part.at[c, pl.ds(s * ROWS_PER_TILE, ROWS_PER_TILE)])


@functools.cache
def _get_layer():
    return pl.kernel(
        _layer_body,
        out_type=jax.ShapeDtypeStruct((NC, N_PAD, D_FEAT), jnp.float32),
        mesh=plsc.VectorSubcoreMesh(core_axis_name="c", subcore_axis_name="s",
                                    num_cores=NC, num_subcores=NS),
        compiler_params=pltpu.CompilerParams(needs_layout_passes=False),
        scratch_types=[
            pltpu.VMEM_SHARED((N_PAD, D_FEAT), jnp.float32),
            pltpu.VMEM((NCH, CK), jnp.int32),
            [pltpu.VMEM((CK,), jnp.int32)] * EDEPTH,
            [pltpu.VMEM((CK,), jnp.float32)] * EDEPTH,
            [pltpu.VMEM((CK, D_FEAT), jnp.float32)] * NBUF,
            [pltpu.SemaphoreType.DMA] * NBUF,
            [pltpu.SemaphoreType.DMA] * NBUF,
            [pltpu.SemaphoreType.DMA] * EDEPTH,
        ],
    )


def _combine_body(p_ref, o_ref):
    o_ref[...] = p_ref[0] + p_ref[1]


def _final_body(x_ref, h1_ref, p_ref, o_ref):
    o_ref[...] = (x_ref[...] + h1_ref[...] + p_ref[0] + p_ref[1]) * (1.0 / 3.0)


_GB = 1000

_combine = pl.pallas_call(
    _combine_body,
    grid=(N_NODES // _GB,),
    in_specs=[pl.BlockSpec((NC, _GB, D_FEAT), lambda i: (0, i, 0))],
    out_specs=pl.BlockSpec((_GB, D_FEAT), lambda i: (i, 0)),
    out_shape=jax.ShapeDtypeStruct((N_NODES, D_FEAT), jnp.float32),
)

_final = pl.pallas_call(
    _final_body,
    grid=(N_NODES // _GB,),
    in_specs=[
        pl.BlockSpec((_GB, D_FEAT), lambda i: (i, 0)),
        pl.BlockSpec((_GB, D_FEAT), lambda i: (i, 0)),
        pl.BlockSpec((NC, _GB, D_FEAT), lambda i: (0, i, 0)),
    ],
    out_specs=pl.BlockSpec((_GB, D_FEAT), lambda i: (i, 0)),
    out_shape=jax.ShapeDtypeStruct((N_NODES, D_FEAT), jnp.float32),
)


def kernel(x, edge_index, edge_weight):
    src = edge_index[0].astype(jnp.int32)
    dst = edge_index[1].astype(jnp.int32)
    ew = edge_weight.astype(jnp.float32)

    pad = E_PAD - N_EDGES
    src = jnp.pad(src, (0, pad))
    dst = jnp.pad(dst, (0, pad)).reshape(NW, NCH, CK)
    ew = jnp.pad(ew, (0, pad))

    zeros = jnp.zeros((ROWS_PER_TILE, D_FEAT), jnp.float32)

    layer = _get_layer()
    p1 = layer(x, src, dst, ew, zeros)
    h1 = _combine(p1)
    p2 = layer(h1, src, dst, ew, zeros)
    return _final(x, h1, p2)
